# baseline (speedup 1.0000x reference)
import functools

import jax
import jax.numpy as jnp
from jax import lax
from jax.experimental import pallas as pl
from jax.experimental.pallas import tpu as pltpu
from jax.experimental.pallas import tpu_sc as plsc

BETA_ = 0.25
TM = 512
CK = 2048


def _vq_tc_kernel(zf_ref, emb_ref, idx_ref, loss_ref, e2_ref):
    i = pl.program_id(0)
    k, d = emb_ref.shape
    nc = k // CK

    @pl.when(i == 0)
    def _():
        e = emb_ref[...]
        e2_ref[...] = jnp.sum(e * e, axis=1)[None, :]
        loss_ref[...] = jnp.zeros((1, 1), jnp.float32)

    z = zf_ref[...]
    x2 = jnp.sum(z * z, axis=1, keepdims=True)
    z2 = z + z
    col = lax.broadcasted_iota(jnp.int32, (TM, CK), 1)

    def dotc(c):
        e_c = emb_ref[c * CK:(c + 1) * CK, :]
        return lax.dot_general(z2, e_c, (((1,), (1,)), ((), ())),
                               preferred_element_type=jnp.float32)

    run_m = None
    run_i = None
    d_prev = dotc(0)
    for c in range(nc):
        d_next = dotc(c + 1) if c + 1 < nc else None
        d2 = x2 - d_prev + e2_ref[:, c * CK:(c + 1) * CK]
        dist = jnp.sqrt(d2)
        mck = jnp.min(dist, axis=1, keepdims=True)
        icl = jnp.min(jnp.where(dist == mck, col, CK), axis=1)
        ic = icl + (c * CK)
        mc = mck[:, 0]
        if c == 0:
            run_m, run_i = mc, ic
        else:
            better = mc < run_m
            run_i = jnp.where(better, ic, run_i)
            run_m = jnp.where(better, mc, run_m)
        d_prev = d_next

    idx_ref[...] = run_i
    loss_ref[...] += jnp.sum(run_m * run_m).reshape(1, 1)


@jax.jit
def _vq_core(z_flat, embedding_weight):
    n, d = z_flat.shape
    k = embedding_weight.shape[0]
    grid = (n // TM,)
    idx, loss_sum = pl.pallas_call(
        _vq_tc_kernel,
        grid=grid,
        in_specs=[
            pl.BlockSpec((TM, d), lambda i: (i, 0)),
            pl.BlockSpec((k, d), lambda i: (0, 0)),
        ],
        out_specs=[
            pl.BlockSpec((TM,), lambda i: (i,)),
            pl.BlockSpec((1, 1), lambda i: (0, 0)),
        ],
        out_shape=[
            jax.ShapeDtypeStruct((n,), jnp.int32),
            jax.ShapeDtypeStruct((1, 1), jnp.float32),
        ],
        scratch_shapes=[pltpu.VMEM((1, k), jnp.float32)],
    )(z_flat, embedding_weight)
    return idx, loss_sum[0, 0]


def _sc_gather(table, idx):
    v, d = table.shape
    b, = idx.shape
    info = plsc.get_sparse_core_info()
    ncores, nsub = info.num_cores, info.num_subcores
    nw = ncores * nsub
    bpw = b // nw
    mesh = plsc.VectorSubcoreMesh(core_axis_name="c", subcore_axis_name="s")

    @functools.partial(
        pl.kernel, mesh=mesh,
        out_type=jax.ShapeDtypeStruct((b, d), jnp.float32),
        scratch_types=[
            pltpu.VMEM((bpw,), jnp.int32),
            pltpu.VMEM((bpw, d), jnp.float32),
            pltpu.SemaphoreType.DMA,
        ],
    )
    def gk(table_hbm, idx_hbm, out_hbm, idx_v, rows_v, sem):
        wid = lax.axis_index("s") * ncores + lax.axis_index("c")
        base = wid * bpw
        pltpu.sync_copy(idx_hbm.at[pl.ds(base, bpw)], idx_v)
        pltpu.async_copy(table_hbm.at[idx_v], rows_v, sem).wait()
        pltpu.sync_copy(rows_v, out_hbm.at[pl.ds(base, bpw)])

    return gk(table, idx)


def kernel(z, embedding_weight):
    zp = jnp.transpose(z, (0, 2, 3, 1))
    z_shape = zp.shape
    z_flat = zp.reshape(-1, embedding_weight.shape[1])
    idx, loss_sum = _vq_core(z_flat, embedding_weight)
    z_q = _sc_gather(embedding_weight, idx).reshape(z_shape)
    n = z_flat.shape[0] * z_flat.shape[1]
    loss = loss_sum / n + BETA_ * (loss_sum / n)
    z_q_st = zp + lax.stop_gradient(z_q - zp)
    z_q_out = jnp.transpose(z_q_st, (0, 3, 1, 2))
    return (z_q_out, loss, (idx, z_flat))


# speedup vs baseline: 1.0343x; 1.0343x over previous
import functools

import jax
import jax.numpy as jnp
from jax import lax
from jax.experimental import pallas as pl
from jax.experimental.pallas import tpu as pltpu
from jax.experimental.pallas import tpu_sc as plsc

BETA_ = 0.25
TM = 1024
CK = 1024


def _vq_tc_kernel(zf_ref, emb_ref, idx_ref, loss_ref, e2_ref):
    i = pl.program_id(0)
    k, d = emb_ref.shape
    nc = k // CK

    @pl.when(i == 0)
    def _():
        e = emb_ref[...]
        e2_ref[...] = jnp.sum(e * e, axis=1)[None, :]
        loss_ref[...] = jnp.zeros((1, 1), jnp.float32)

    z = zf_ref[...]
    x2 = jnp.sum(z * z, axis=1, keepdims=True)
    z2 = z + z
    col = lax.broadcasted_iota(jnp.int32, (TM, CK), 1)

    def dotc(c):
        e_c = emb_ref[c * CK:(c + 1) * CK, :]
        return lax.dot_general(z2, e_c, (((1,), (1,)), ((), ())),
                               preferred_element_type=jnp.float32)

    run_m = None
    run_i = None
    d_prev = dotc(0)
    for c in range(nc):
        d_next = dotc(c + 1) if c + 1 < nc else None
        d2 = x2 - d_prev + e2_ref[:, c * CK:(c + 1) * CK]
        dist = jnp.sqrt(d2)
        mck = jnp.min(dist, axis=1, keepdims=True)
        icl = jnp.min(jnp.where(dist == mck, col, CK), axis=1)
        ic = icl + (c * CK)
        mc = mck[:, 0]
        if c == 0:
            run_m, run_i = mc, ic
        else:
            better = mc < run_m
            run_i = jnp.where(better, ic, run_i)
            run_m = jnp.where(better, mc, run_m)
        d_prev = d_next

    idx_ref[...] = run_i
    loss_ref[...] += jnp.sum(run_m * run_m).reshape(1, 1)


@jax.jit
def _vq_core(z_flat, embedding_weight):
    n, d = z_flat.shape
    k = embedding_weight.shape[0]
    grid = (n // TM,)
    idx, loss_sum = pl.pallas_call(
        _vq_tc_kernel,
        grid=grid,
        in_specs=[
            pl.BlockSpec((TM, d), lambda i: (i, 0)),
            pl.BlockSpec((k, d), lambda i: (0, 0)),
        ],
        out_specs=[
            pl.BlockSpec((TM,), lambda i: (i,)),
            pl.BlockSpec((1, 1), lambda i: (0, 0)),
        ],
        out_shape=[
            jax.ShapeDtypeStruct((n,), jnp.int32),
            jax.ShapeDtypeStruct((1, 1), jnp.float32),
        ],
        scratch_shapes=[pltpu.VMEM((1, k), jnp.float32)],
    )(z_flat, embedding_weight)
    return idx, loss_sum[0, 0]


def _sc_gather(table, idx):
    v, d = table.shape
    b, = idx.shape
    info = plsc.get_sparse_core_info()
    ncores, nsub = info.num_cores, info.num_subcores
    nw = ncores * nsub
    bpw = b // nw
    mesh = plsc.VectorSubcoreMesh(core_axis_name="c", subcore_axis_name="s")

    @functools.partial(
        pl.kernel, mesh=mesh,
        out_type=jax.ShapeDtypeStruct((b, d), jnp.float32),
        scratch_types=[
            pltpu.VMEM((bpw,), jnp.int32),
            pltpu.VMEM((bpw, d), jnp.float32),
            pltpu.SemaphoreType.DMA,
        ],
    )
    def gk(table_hbm, idx_hbm, out_hbm, idx_v, rows_v, sem):
        wid = lax.axis_index("s") * ncores + lax.axis_index("c")
        base = wid * bpw
        pltpu.sync_copy(idx_hbm.at[pl.ds(base, bpw)], idx_v)
        pltpu.async_copy(table_hbm.at[idx_v], rows_v, sem).wait()
        pltpu.sync_copy(rows_v, out_hbm.at[pl.ds(base, bpw)])

    return gk(table, idx)


def kernel(z, embedding_weight):
    zp = jnp.transpose(z, (0, 2, 3, 1))
    z_shape = zp.shape
    z_flat = zp.reshape(-1, embedding_weight.shape[1])
    idx, loss_sum = _vq_core(z_flat, embedding_weight)
    z_q = _sc_gather(embedding_weight, idx).reshape(z_shape)
    n = z_flat.shape[0] * z_flat.shape[1]
    loss = loss_sum / n + BETA_ * (loss_sum / n)
    z_q_st = zp + lax.stop_gradient(z_q - zp)
    z_q_out = jnp.transpose(z_q_st, (0, 3, 1, 2))
    return (z_q_out, loss, (idx, z_flat))
